# baseline (speedup 1.0000x reference)
import functools

import jax
import jax.numpy as jnp
from jax import lax
from jax.experimental import pallas as pl
from jax.experimental.pallas import tpu as pltpu
from jax.experimental.pallas import tpu_sc as plsc

_VOCAB = 1000000
_BATCH = 16384
_HIST = 50
_D = 64
_NC = 2
_NS = 16
_NW = _NC * _NS
_VBLK = _VOCAB // 128
_JB = _BATCH // 128
_JPW = _JB // _NW
_EPS = 1e-5
_MAXN = 1.0 - _EPS
_THR = _MAXN * _MAXN

def _i16():
    return lax.iota(jnp.int32, 16)


def _rsqrt16(a):
    xi = plsc.bitcast(a, jnp.int32)
    y = plsc.bitcast(jnp.int32(0x5F3759DF) - (xi >> 1), jnp.float32)
    for _ in range(4):
        y = y * (1.5 - 0.5 * a * y * y)
    return y


def _pack_body(tt_hbm, tail_hbm, out_hbm, in0, in1, ob0, ob1, is0, is1, os0, os1):
    wid = lax.axis_index("s") * _NC + lax.axis_index("c")
    inb = (in0, in1)
    outb = (ob0, ob1)
    isem = (is0, is1)
    osem = (os0, os1)
    nk = (_VBLK - wid + _NW - 1) // _NW

    def fire_in(k, b):
        j = wid + k * _NW
        pltpu.async_copy(tt_hbm.at[pl.ds(0, 32), pl.ds(j * 128, 128)],
                         inb[b], isem[b])

    def drain_in(b):
        pltpu.make_async_copy(tt_hbm.at[pl.ds(0, 32), pl.ds(0, 128)],
                              inb[b], isem[b]).wait()

    def fire_out(k, b):
        j = wid + k * _NW
        pltpu.async_copy(outb[b], out_hbm.at[pl.ds(j * 32, 32)], osem[b])

    def drain_out(b):
        pltpu.make_async_copy(outb[b], out_hbm.at[pl.ds(0, 32)], osem[b]).wait()

    iota = _i16()
    idx0 = [iota + 16 * a for a in range(2)]

    def transpose(b):
        @plsc.parallel_loop(0, 128, unroll=4)
        def _(v):
            vl = (jnp.full((16,), v, jnp.int32) + iota) & 127
            vh = vl >> 2
            vo = (vl & 3) * 32
            for a in range(2):
                vec = plsc.load_gather(inb[b], [idx0[a], vl])
                plsc.store_scatter(outb[b], [vh, vo + idx0[a]], vec)

    @pl.when(nk > 0)
    def _():
        fire_in(0, 0)

    def body(i, carry):
        for b in range(2):
            k = i * 2 + b

            @pl.when(k < nk)
            def _():
                drain_in(b)

                @pl.when(k >= 1)
                def _():
                    drain_out(1 - b)

                @pl.when(k + 1 < nk)
                def _():
                    fire_in(k + 1, 1 - b)

                transpose(b)
                fire_out(k, b)
        return carry

    lax.fori_loop(0, (nk + 1) // 2, body, 0)
    @pl.when(nk > 0)
    def _():
        @pl.when(nk % 2 == 1)
        def _():
            drain_out(0)

        @pl.when(nk % 2 == 0)
        def _():
            drain_out(1)

    @pl.when(wid == (_VBLK % _NW))
    def _():
        pltpu.sync_copy(tail_hbm, ob0.at[pl.ds(0, 16)])
        pltpu.sync_copy(ob0.at[pl.ds(0, 16)],
                        out_hbm.at[pl.ds(_VBLK * 32, 16)])


_mesh = plsc.VectorSubcoreMesh(core_axis_name="c", subcore_axis_name="s")

_pack = functools.partial(
    pl.kernel,
    mesh=_mesh,
    out_type=jax.ShapeDtypeStruct((_VOCAB // 4, 128), jnp.float32),
    scratch_types=[
        pltpu.VMEM((32, 128), jnp.float32),
        pltpu.VMEM((32, 128), jnp.float32),
        pltpu.VMEM((32, 128), jnp.float32),
        pltpu.VMEM((32, 128), jnp.float32),
        pltpu.SemaphoreType.DMA,
        pltpu.SemaphoreType.DMA,
        pltpu.SemaphoreType.DMA,
        pltpu.SemaphoreType.DMA,
    ],
    compiler_params=pltpu.CompilerParams(use_tc_tiling_on_sc=True, needs_layout_passes=False),
)(_pack_body)


def _gather_body(ext_hbm, tlo_hbm, thi_hbm, out_hbm, idx_v, l0, l1, hh0, hh1,
                 t0, t1, is0, is1, os0, os1):
    wid = lax.axis_index("s") * _NC + lax.axis_index("c")
    rlo = (l0, l1)
    rhi = (hh0, hh1)
    tb = (t0, t1)
    isem = (is0, is1)
    osem = (os0, os1)
    nblk = _HIST * _JPW

    pltpu.sync_copy(ext_hbm.at[:, pl.ds(wid * (128 * _JPW), 128 * _JPW)], idx_v)

    def fire_in(g, b):
        h = g >> 2
        jj = g & 3
        iref = idx_v.at[h, pl.ds(jj * 128, 128)]
        pltpu.async_copy(tlo_hbm.at[iref], rlo[b], isem[b])
        pltpu.async_copy(thi_hbm.at[iref], rhi[b], isem[b])

    def drain_in(b):
        pltpu.make_async_copy(tlo_hbm.at[pl.ds(0, 128)], rlo[b],
                              isem[b]).wait()
        pltpu.make_async_copy(thi_hbm.at[pl.ds(0, 128)], rhi[b],
                              isem[b]).wait()

    def fire_out(g, b):
        h = g >> 2
        j = wid * _JPW + (g & 3)
        for i in range(8):
            pltpu.async_copy(tb[b].at[pl.ds(8 * i, 8)], out_hbm.at[h, i, j],
                             osem[b])

    def drain_out(b):
        for i in range(8):
            pltpu.make_async_copy(tb[b].at[pl.ds(8 * i, 8)],
                                  out_hbm.at[0, 0, 0], osem[b]).wait()

    iota = _i16()
    idxs = [iota + 16 * s for s in range(8)]

    def transpose_norm(b):
        zeros = jnp.zeros((16,), jnp.float32)

        @plsc.parallel_loop(0, 32, unroll=4, carry=(zeros,) * 8)
        def frow(f, acc):
            cv = (jnp.full((16,), f, jnp.int32) + iota) & 31
            new = []
            for s in range(8):
                vlo = plsc.load_gather(rlo[b], [idxs[s], cv])
                plsc.store_scatter(tb[b], [cv, idxs[s]], vlo)
                vhi = plsc.load_gather(rhi[b], [idxs[s], cv])
                plsc.store_scatter(tb[b], [cv + 32, idxs[s]], vhi)
                new.append(acc[s] + vlo * vlo + vhi * vhi)
            return tuple(new)

        acc = frow
        hot = acc[0] > _THR
        for s in range(1, 8):
            hot = hot | (acc[s] > _THR)
        need = jnp.any(hot)

        @pl.when(need)
        def _():
            scales = [
                jnp.where(acc[s] > _THR, _MAXN * _rsqrt16(acc[s]), 1.0)
                for s in range(8)
            ]

            def fix(f, carry):
                for s in range(8):
                    sl = tb[b].at[f, pl.ds(16 * s, 16)]
                    sl[...] = sl[...] * scales[s]
                return carry

            lax.fori_loop(0, _D, fix, 0)

    fire_in(0, 0)

    def body(i, carry):
        for b in range(2):
            g = i * 2 + b
            drain_in(b)

            @pl.when(g >= 1)
            def _():
                drain_out(1 - b)

            @pl.when(g + 1 < nblk)
            def _():
                fire_in(g + 1, 1 - b)

            transpose_norm(b)
            fire_out(g, b)
        return carry

    lax.fori_loop(0, nblk // 2, body, 0)
    drain_out(1)


_gather = functools.partial(
    pl.kernel,
    mesh=_mesh,
    out_type=jax.ShapeDtypeStruct((_HIST, 8, _JB, 8, 128), jnp.float32),
    scratch_types=[
        pltpu.VMEM((_HIST, 128 * _JPW), jnp.int32),
        pltpu.VMEM((128, 32), jnp.float32),
        pltpu.VMEM((128, 32), jnp.float32),
        pltpu.VMEM((128, 32), jnp.float32),
        pltpu.VMEM((128, 32), jnp.float32),
        pltpu.VMEM((_D, 128), jnp.float32),
        pltpu.VMEM((_D, 128), jnp.float32),
        pltpu.SemaphoreType.DMA,
        pltpu.SemaphoreType.DMA,
        pltpu.SemaphoreType.DMA,
        pltpu.SemaphoreType.DMA,
    ],
    compiler_params=pltpu.CompilerParams(use_tc_tiling_on_sc=False, needs_layout_passes=False),
)(_gather_body)


def kernel(examples, table):
    tail = lax.slice(table, (_VBLK * 128, 0), (_VOCAB, 32)).reshape(16, 128)
    packed = _pack(table.T, tail)
    tlo = packed.reshape(_VOCAB, 32)
    thi = lax.slice(table, (0, 32), (_VOCAB, _D))
    ext = examples.T
    out5 = _gather(ext, tlo, thi)
    return out5.transpose(2, 4, 0, 1, 3).reshape(_BATCH, _HIST, _D)


# speedup vs baseline: 1.8003x; 1.8003x over previous
import functools

import jax
import jax.numpy as jnp
from jax import lax
from jax.experimental import pallas as pl
from jax.experimental.pallas import tpu as pltpu
from jax.experimental.pallas import tpu_sc as plsc

_VOCAB = 1000000
_BATCH = 16384
_HIST = 50
_D = 64
_NC = 2
_NS = 16
_NW = _NC * _NS
_VBLK = _VOCAB // 128
_JB = _BATCH // 128
_JPW = _JB // _NW
_EPS = 1e-5
_MAXN = 1.0 - _EPS
_THR = _MAXN * _MAXN

def _i16():
    return lax.iota(jnp.int32, 16)


def _rsqrt16(a):
    xi = plsc.bitcast(a, jnp.int32)
    y = plsc.bitcast(jnp.int32(0x5F3759DF) - (xi >> 1), jnp.float32)
    for _ in range(4):
        y = y * (1.5 - 0.5 * a * y * y)
    return y


def _pack_body(tt_hbm, tail_hbm, out_hbm, in0, in1, ob0, ob1, is0, is1, os0, os1):
    wid = lax.axis_index("s") * _NC + lax.axis_index("c")
    inb = (in0, in1)
    outb = (ob0, ob1)
    isem = (is0, is1)
    osem = (os0, os1)
    nk = (_VBLK - wid + _NW - 1) // _NW

    def fire_in(k, b):
        j = wid + k * _NW
        pltpu.async_copy(tt_hbm.at[:, pl.ds(j * 128, 128)], inb[b], isem[b])

    def drain_in(b):
        pltpu.make_async_copy(tt_hbm.at[:, pl.ds(0, 128)], inb[b], isem[b]).wait()

    def fire_out(k, b):
        j = wid + k * _NW
        pltpu.async_copy(outb[b], out_hbm.at[pl.ds(j * 64, 64)], osem[b])

    def drain_out(b):
        pltpu.make_async_copy(outb[b], out_hbm.at[pl.ds(0, 64)], osem[b]).wait()

    iota = _i16()
    idx0 = [iota + 16 * a for a in range(4)]

    def transpose(b):
        @plsc.parallel_loop(0, 128, unroll=4)
        def _(v):
            vl = (jnp.full((16,), v, jnp.int32) + iota) & 127
            vh = vl >> 1
            vo = (vl & 1) * 64
            for a in range(4):
                vec = plsc.load_gather(inb[b], [idx0[a], vl])
                plsc.store_scatter(outb[b], [vh, vo + idx0[a]], vec)

    @pl.when(nk > 0)
    def _():
        fire_in(0, 0)

    def body(i, carry):
        for b in range(2):
            k = i * 2 + b

            @pl.when(k < nk)
            def _():
                drain_in(b)

                @pl.when(k >= 1)
                def _():
                    drain_out(1 - b)

                @pl.when(k + 1 < nk)
                def _():
                    fire_in(k + 1, 1 - b)

                transpose(b)
                fire_out(k, b)
        return carry

    lax.fori_loop(0, (nk + 1) // 2, body, 0)
    @pl.when(nk > 0)
    def _():
        @pl.when(nk % 2 == 1)
        def _():
            drain_out(0)

        @pl.when(nk % 2 == 0)
        def _():
            drain_out(1)

    @pl.when(wid == (_VBLK % _NW))
    def _():
        pltpu.sync_copy(tail_hbm, ob0.at[pl.ds(0, 32)])
        pltpu.sync_copy(ob0.at[pl.ds(0, 32)],
                        out_hbm.at[pl.ds(_VBLK * 64, 32)])


_mesh = plsc.VectorSubcoreMesh(core_axis_name="c", subcore_axis_name="s")

_pack = functools.partial(
    pl.kernel,
    mesh=_mesh,
    out_type=jax.ShapeDtypeStruct((_VOCAB // 2, 128), jnp.float32),
    scratch_types=[
        pltpu.VMEM((_D, 128), jnp.float32),
        pltpu.VMEM((_D, 128), jnp.float32),
        pltpu.VMEM((_D, 128), jnp.float32),
        pltpu.VMEM((_D, 128), jnp.float32),
        pltpu.SemaphoreType.DMA,
        pltpu.SemaphoreType.DMA,
        pltpu.SemaphoreType.DMA,
        pltpu.SemaphoreType.DMA,
    ],
    compiler_params=pltpu.CompilerParams(use_tc_tiling_on_sc=True, needs_layout_passes=False),
)(_pack_body)


def _gather_body(ext_hbm, table_hbm, out_hbm, idx_v, r0, r1, t0, t1,
                 is0, is1, os0, os1):
    wid = lax.axis_index("s") * _NC + lax.axis_index("c")
    rows = (r0, r1)
    tb = (t0, t1)
    isem = (is0, is1)
    osem = (os0, os1)
    nblk = _HIST * _JPW

    pltpu.sync_copy(ext_hbm.at[:, pl.ds(wid * (128 * _JPW), 128 * _JPW)], idx_v)

    def fire_in(g, b):
        h = g >> 2
        jj = g & 3
        pltpu.async_copy(table_hbm.at[idx_v.at[h, pl.ds(jj * 128, 128)]],
                         rows[b], isem[b])

    def drain_in(b):
        pltpu.make_async_copy(table_hbm.at[pl.ds(0, 128)], rows[b],
                              isem[b]).wait()

    def fire_out(g, b):
        h = g >> 2
        j = wid * _JPW + (g & 3)
        for i in range(8):
            pltpu.async_copy(tb[b].at[pl.ds(8 * i, 8)], out_hbm.at[h, i, j],
                             osem[b])

    def drain_out(b):
        for i in range(8):
            pltpu.make_async_copy(tb[b].at[pl.ds(8 * i, 8)],
                                  out_hbm.at[0, 0, 0], osem[b]).wait()

    iota = _i16()
    idxs = [iota + 16 * s for s in range(8)]

    def transpose_norm(b):
        zeros = jnp.zeros((16,), jnp.float32)

        @plsc.parallel_loop(0, _D, unroll=4, carry=(zeros,) * 8)
        def frow(f, acc):
            cv = (jnp.full((16,), f, jnp.int32) + iota) & 63
            new = []
            for s in range(8):
                vec = plsc.load_gather(rows[b], [idxs[s], cv])
                plsc.store_scatter(tb[b], [cv, idxs[s]], vec)
                new.append(acc[s] + vec * vec)
            return tuple(new)

        acc = frow
        hot = acc[0] > _THR
        for s in range(1, 8):
            hot = hot | (acc[s] > _THR)
        need = jnp.any(hot)

        @pl.when(need)
        def _():
            scales = [
                jnp.where(acc[s] > _THR, _MAXN * _rsqrt16(acc[s]), 1.0)
                for s in range(8)
            ]

            def fix(f, carry):
                for s in range(8):
                    sl = tb[b].at[f, pl.ds(16 * s, 16)]
                    sl[...] = sl[...] * scales[s]
                return carry

            lax.fori_loop(0, _D, fix, 0)

    fire_in(0, 0)

    def body(i, carry):
        for b in range(2):
            g = i * 2 + b
            drain_in(b)

            @pl.when(g >= 1)
            def _():
                drain_out(1 - b)

            @pl.when(g + 1 < nblk)
            def _():
                fire_in(g + 1, 1 - b)

            transpose_norm(b)
            fire_out(g, b)
        return carry

    lax.fori_loop(0, nblk // 2, body, 0)
    drain_out(1)


_gather = functools.partial(
    pl.kernel,
    mesh=_mesh,
    out_type=jax.ShapeDtypeStruct((_HIST, 8, _JB, 8, 128), jnp.float32),
    scratch_types=[
        pltpu.VMEM((_HIST, 128 * _JPW), jnp.int32),
        pltpu.VMEM((128, _D), jnp.float32),
        pltpu.VMEM((128, _D), jnp.float32),
        pltpu.VMEM((_D, 128), jnp.float32),
        pltpu.VMEM((_D, 128), jnp.float32),
        pltpu.SemaphoreType.DMA,
        pltpu.SemaphoreType.DMA,
        pltpu.SemaphoreType.DMA,
        pltpu.SemaphoreType.DMA,
    ],
    compiler_params=pltpu.CompilerParams(use_tc_tiling_on_sc=False, needs_layout_passes=False),
)(_gather_body)


def kernel(examples, table):
    tail = lax.slice(table, (_VBLK * 128, 0), (_VOCAB, _D)).reshape(32, 128)
    packed = _pack(table.T, tail)
    tlin = packed.reshape(_VOCAB, _D)
    ext = examples.T
    out5 = _gather(ext, tlin)
    return out5.transpose(2, 4, 0, 1, 3).reshape(_BATCH, _HIST, _D)
